# baseline (speedup 1.0000x reference)
import functools

import jax
import jax.numpy as jnp
from jax.experimental import pallas as pl
from jax.experimental.pallas import tpu as pltpu

_B, _A, _H, _W, _NC = 16, 3, 80, 80, 80
_NT = 200
_M = _A * _NT
_MP = 640
_ROWS = _B * _A * _H * _W
_C = 5 + _NC
_OBJ_BLK = 2048
_N_OBJ_BLKS = _ROWS // _OBJ_BLK


def _softplus(x):
    return jnp.maximum(x, 0.0) + jnp.log1p(jnp.exp(-jnp.abs(x)))


def _obj_body(x_ref, o_ref, acc_ref):
    i = pl.program_id(0)

    @pl.when(i == 0)
    def _():
        acc_ref[0] = 0.0

    acc_ref[0] += jnp.sum(_softplus(x_ref[:, 4:5]))

    @pl.when(i == pl.num_programs(0) - 1)
    def _():
        o_ref[...] = jnp.full((1, 1), acc_ref[0], jnp.float32)


def _gather_body(idx_ref, x_ref, o_ref):
    o_ref[...] = x_ref[...]


def _loss_body(t_ref, tt_ref, ps_ref, anch_ref, s_ref, objsum_ref, o_ref):
    s = s_ref[0]
    gain = jnp.float32(_W)

    def cand_cols(a):
        anc_w = anch_ref[a, 0] / s
        anc_h = anch_ref[a, 1] / s
        bi = t_ref[:, 0:1].astype(jnp.int32)
        cls = t_ref[:, 1:2].astype(jnp.int32)
        gx = t_ref[:, 2:3] * gain
        gy = t_ref[:, 3:4] * gain
        gw = t_ref[:, 4:5] * gain
        gh = t_ref[:, 5:6] * gain
        rw = gw / anc_w
        rh = gh / anc_h
        mask = jnp.logical_and(jnp.maximum(rw, 1.0 / rw) < 4.0,
                               jnp.maximum(rh, 1.0 / rh) < 4.0)
        fx = gx.astype(jnp.int32)
        fy = gy.astype(jnp.int32)
        gi = jnp.clip(fx, 0, _W - 1)
        gj = jnp.clip(fy, 0, _H - 1)
        row = ((bi * _A + a) * _H + gj) * _W + gi
        tbx = gx - fx.astype(jnp.float32)
        tby = gy - fy.astype(jnp.float32)
        return (mask.astype(jnp.float32), row, tbx, tby, gw, gh, cls,
                jnp.full((_NT, 1), anc_w, jnp.float32),
                jnp.full((_NT, 1), anc_h, jnp.float32))

    def catpad(parts, padval, dtype):
        pad = jnp.full((_MP - _M, 1), padval, dtype)
        return jnp.concatenate(list(parts) + [pad], axis=0)

    c0, c1, c2 = cand_cols(0), cand_cols(1), cand_cols(2)
    mf = catpad([c0[0], c1[0], c2[0]], 0.0, jnp.float32)
    row = catpad([c0[1], c1[1], c2[1]], -1, jnp.int32)
    tbx = catpad([c0[2], c1[2], c2[2]], 0.0, jnp.float32)
    tby = catpad([c0[3], c1[3], c2[3]], 0.0, jnp.float32)
    tbw = catpad([c0[4], c1[4], c2[4]], 0.0, jnp.float32)
    tbh = catpad([c0[5], c1[5], c2[5]], 0.0, jnp.float32)
    cls = catpad([c0[6], c1[6], c2[6]], 0, jnp.int32)
    anw = catpad([c0[7], c1[7], c2[7]], 1.0, jnp.float32)
    anh = catpad([c0[8], c1[8], c2[8]], 1.0, jnp.float32)

    def cand_rows(a):
        anc_w = anch_ref[a, 0] / s
        anc_h = anch_ref[a, 1] / s
        bi = tt_ref[0:1, :].astype(jnp.int32)
        gx = tt_ref[2:3, :] * gain
        gy = tt_ref[3:4, :] * gain
        gw = tt_ref[4:5, :] * gain
        gh = tt_ref[5:6, :] * gain
        rw = gw / anc_w
        rh = gh / anc_h
        mask = jnp.logical_and(jnp.maximum(rw, 1.0 / rw) < 4.0,
                               jnp.maximum(rh, 1.0 / rh) < 4.0)
        gi = jnp.clip(gx.astype(jnp.int32), 0, _W - 1)
        gj = jnp.clip(gy.astype(jnp.int32), 0, _H - 1)
        rowr = ((bi * _A + a) * _H + gj) * _W + gi
        return mask.astype(jnp.float32), rowr

    r0, r1, r2 = cand_rows(0), cand_rows(1), cand_rows(2)
    padm = jnp.zeros((1, _MP - _M), jnp.float32)
    padr = jnp.full((1, _MP - _M), -2, jnp.int32)
    mf_r = jnp.concatenate([r0[0], r1[0], r2[0], padm], axis=1)
    row_r = jnp.concatenate([r0[1], r1[1], r2[1], padr], axis=1)

    kk = jax.lax.broadcasted_iota(jnp.int32, (_MP, _MP), 0)
    jj = jax.lax.broadcasted_iota(jnp.int32, (_MP, _MP), 1)
    later_dup = (row == row_r).astype(jnp.float32) * mf_r * (jj > kk).astype(
        jnp.float32)
    ndup = jnp.sum(later_dup, axis=1, keepdims=True)
    winner = mf * (ndup < 0.5).astype(jnp.float32)

    pxy_x = 1.0 / (1.0 + jnp.exp(-ps_ref[:, 0:1]))
    pxy_y = 1.0 / (1.0 + jnp.exp(-ps_ref[:, 1:2]))
    pw = jnp.exp(ps_ref[:, 2:3]) * anw
    ph = jnp.exp(ps_ref[:, 3:4]) * anh
    p4 = ps_ref[:, 4:5]

    b1x1 = pxy_x - pw * 0.5
    b1x2 = pxy_x + pw * 0.5
    b1y1 = pxy_y - ph * 0.5
    b1y2 = pxy_y + ph * 0.5
    b2x1 = tbx - tbw * 0.5
    b2x2 = tbx + tbw * 0.5
    b2y1 = tby - tbh * 0.5
    b2y2 = tby + tbh * 0.5
    iw = jnp.maximum(jnp.minimum(b1x2, b2x2) - jnp.maximum(b1x1, b2x1), 0.0)
    ih = jnp.maximum(jnp.minimum(b1y2, b2y2) - jnp.maximum(b1y1, b2y1), 0.0)
    inter = iw * ih
    union = pw * ph + tbw * tbh - inter + 1e-9
    iou = inter / union

    msum = jnp.sum(mf)
    denom = jnp.maximum(msum, 1.0)
    has = (msum > 0.0).astype(jnp.float32)
    lbox = has * jnp.sum((1.0 - iou) * mf) / denom

    logits = ps_ref[:, 5:_C]
    cc = jax.lax.broadcasted_iota(jnp.int32, (_MP, _NC), 1)
    sel = jnp.sum(logits * (cc == cls).astype(jnp.float32), axis=1,
                  keepdims=True)
    spsum = jnp.sum(_softplus(logits), axis=1, keepdims=True)
    lcls = has * jnp.sum((spsum - sel) * mf) / (denom * _NC)

    corr = jnp.sum(winner * p4 * jnp.maximum(iou, 0.0))
    lobj = (objsum_ref[0, 0] - corr) / jnp.float32(_ROWS)

    o_ref[0, 0] = 0.05 * lcls + lobj + 0.5 * lbox


def _cand_rows_host(targets, anchors, stride):
    anc = anchors / stride
    gain = jnp.float32(_W)
    gx = targets[:, 2] * gain
    gy = targets[:, 3] * gain
    gi = jnp.clip(gx.astype(jnp.int32), 0, _W - 1)
    gj = jnp.clip(gy.astype(jnp.int32), 0, _H - 1)
    bi = targets[:, 0].astype(jnp.int32)
    rows = (bi[None, :] * _A + jnp.arange(_A, dtype=jnp.int32)[:, None])
    rows = (rows * _H + gj[None, :]) * _W + gi[None, :]
    rows = rows.reshape(-1)
    return jnp.concatenate([rows, jnp.zeros((_MP - _M,), jnp.int32)])


@jax.jit
def kernel(pred, targets, anchors, stride):
    pred2d = pred.reshape(_ROWS, _C)

    objsum = pl.pallas_call(
        _obj_body,
        grid=(_N_OBJ_BLKS,),
        in_specs=[pl.BlockSpec((_OBJ_BLK, _C), lambda i: (i, 0))],
        out_specs=pl.BlockSpec((1, 1), lambda i: (0, 0)),
        out_shape=jax.ShapeDtypeStruct((1, 1), jnp.float32),
        scratch_shapes=[pltpu.SMEM((1,), jnp.float32)],
    )(pred2d)

    rows = _cand_rows_host(targets, anchors, stride)
    pred3d = pred2d.reshape(_ROWS, 1, _C)
    ps = pl.pallas_call(
        _gather_body,
        grid_spec=pltpu.PrefetchScalarGridSpec(
            num_scalar_prefetch=1,
            grid=(_MP,),
            in_specs=[pl.BlockSpec((1, 1, _C), lambda i, idx: (idx[i], 0, 0))],
            out_specs=pl.BlockSpec((1, 1, _C), lambda i, idx: (i, 0, 0)),
        ),
        out_shape=jax.ShapeDtypeStruct((_MP, 1, _C), jnp.float32),
    )(rows, pred3d)

    loss = pl.pallas_call(
        _loss_body,
        in_specs=[
            pl.BlockSpec(memory_space=pltpu.VMEM),
            pl.BlockSpec(memory_space=pltpu.VMEM),
            pl.BlockSpec(memory_space=pltpu.VMEM),
            pl.BlockSpec(memory_space=pltpu.SMEM),
            pl.BlockSpec(memory_space=pltpu.SMEM),
            pl.BlockSpec(memory_space=pltpu.SMEM),
        ],
        out_specs=pl.BlockSpec(memory_space=pltpu.SMEM),
        out_shape=jax.ShapeDtypeStruct((1, 1), jnp.float32),
    )(targets, targets.T, ps.reshape(_MP, _C), anchors,
      stride.reshape(1), objsum)

    return loss.reshape(())


# speedup vs baseline: 1.9538x; 1.9538x over previous
import functools

import jax
import jax.numpy as jnp
from jax import lax
from jax.experimental import pallas as pl
from jax.experimental.pallas import tpu as pltpu
from jax.experimental.pallas import tpu_sc as plsc

_B, _A, _H, _W, _NC = 16, 3, 80, 80, 80
_NT = 200
_NTP = 208
_M = _A * _NT
_MP = 640
_ROWS = _B * _A * _H * _W
_C = 5 + _NC
_CW = 88

_NTILES = 32
_NGRP = _MP // 16
_GPW = 16 * _CW
_OPT = _ROWS // _NTILES
_OCHUNKS = _OPT // 128
_GCHUNKS = _GPW // 128


def _softplus(x):
    return jnp.maximum(x, 0.0) + jnp.log1p(jnp.exp(-jnp.abs(x)))


def _step01(x):
    return jnp.minimum(jnp.maximum(x, 0), 1)


def _sc_body(pred_hbm, targ_hbm, obj_hbm, rows_hbm,
             idxo, objbuf, idx2, rowbuf, tvm, sem_o, sem_r, sem_t):
    wid = lax.axis_index("s") * 2 + lax.axis_index("c")
    lanes = lax.iota(jnp.int32, 16)

    pltpu.async_copy(targ_hbm, tvm, sem_t).wait()

    base_o = wid * _OPT

    def gen_obj(j, carry):
        idxo[pl.ds(j * 16, 16)] = (base_o + j * 16 + lanes) * _C + 4
        return carry

    lax.fori_loop(0, _OPT // 16, gen_obj, 0)

    def fire_o(r, carry):
        pltpu.async_copy(pred_hbm.at[idxo.at[pl.ds(r * 128, 128)]],
                         objbuf.at[pl.ds(r * 128, 128)], sem_o)
        return carry

    lax.fori_loop(0, _OCHUNKS, fire_o, 0)

    jbs = [jnp.minimum(c * 16 + lanes, _C - 1) for c in range(6)]

    def do_group(gg, slot):
        a3 = (_step01(gg - 12) + _step01(gg - 25) + _step01(gg - 38))
        a_c = jnp.minimum(a3, 2)
        i0 = (gg - 13 * a3) * 16
        bf = tvm[pl.ds(i0, 16)]
        xf = tvm[pl.ds(2 * _NTP + i0, 16)]
        yf = tvm[pl.ds(3 * _NTP + i0, 16)]
        gi = jnp.clip((xf * jnp.float32(_W)).astype(jnp.int32), 0, _W - 1)
        gj = jnp.clip((yf * jnp.float32(_H)).astype(jnp.int32), 0, _H - 1)
        bi = bf.astype(jnp.int32)
        row85 = (((bi * _A + a_c) * _H + gj) * _W + gi) * _C
        buf_base = slot * _GPW
        for kk in range(16):
            rv = row85[kk]
            for c in range(6):
                idx2[pl.ds(buf_base + kk * _CW + c * 16, 16)] = jbs[c] + rv

        def fire_r(r, carry):
            pltpu.async_copy(
                pred_hbm.at[idx2.at[pl.ds(buf_base + r * 128, 128)]],
                rowbuf.at[pl.ds(buf_base + r * 128, 128)], sem_r)
            return carry

        lax.fori_loop(0, _GCHUNKS, fire_r, 0)

    do_group(wid, 0)

    @pl.when(wid < _NGRP - _NTILES)
    def _():
        do_group(wid + _NTILES, 1)

    pltpu.make_async_copy(pred_hbm.at[pl.ds(0, _OPT)], objbuf, sem_o).wait()

    pltpu.make_async_copy(pred_hbm.at[pl.ds(0, _GPW)],
                          rowbuf.at[pl.ds(0, _GPW)], sem_r).wait()

    @pl.when(wid < _NGRP - _NTILES)
    def _():
        pltpu.make_async_copy(pred_hbm.at[pl.ds(0, _GPW)],
                              rowbuf.at[pl.ds(_GPW, _GPW)], sem_r).wait()

    pltpu.sync_copy(objbuf, obj_hbm.at[wid])
    pltpu.sync_copy(rowbuf.at[pl.ds(0, _GPW)], rows_hbm.at[wid])

    @pl.when(wid < _NGRP - _NTILES)
    def _():
        pltpu.sync_copy(rowbuf.at[pl.ds(_GPW, _GPW)],
                        rows_hbm.at[wid + _NTILES])


def _sc_gather(predflat, targt):
    mesh = plsc.VectorSubcoreMesh(core_axis_name="c", subcore_axis_name="s")
    return pl.kernel(
        _sc_body,
        out_type=(
            jax.ShapeDtypeStruct((_NTILES, _OPT), jnp.float32),
            jax.ShapeDtypeStruct((_NGRP, _GPW), jnp.float32),
        ),
        mesh=mesh,
        scratch_types=[
            pltpu.VMEM((_OPT,), jnp.int32),
            pltpu.VMEM((_OPT,), jnp.float32),
            pltpu.VMEM((2 * _GPW + 16,), jnp.int32),
            pltpu.VMEM((2 * _GPW,), jnp.float32),
            pltpu.VMEM((6 * _NTP,), jnp.float32),
            pltpu.SemaphoreType.DMA,
            pltpu.SemaphoreType.DMA,
            pltpu.SemaphoreType.DMA,
        ],
    )(predflat, targt)


def _loss_body(obj_ref, ps_ref, t_ref, tt_ref, anch_ref, s_ref, o_ref):
    s = s_ref[0]
    gain = jnp.float32(_W)

    def cand_cols(a):
        anc_w = anch_ref[a, 0] / s
        anc_h = anch_ref[a, 1] / s
        bi = t_ref[:, 0:1].astype(jnp.int32)
        cls = t_ref[:, 1:2].astype(jnp.int32)
        gx = t_ref[:, 2:3] * gain
        gy = t_ref[:, 3:4] * gain
        gw = t_ref[:, 4:5] * gain
        gh = t_ref[:, 5:6] * gain
        rw = gw / anc_w
        rh = gh / anc_h
        mask = jnp.logical_and(jnp.maximum(rw, 1.0 / rw) < 4.0,
                               jnp.maximum(rh, 1.0 / rh) < 4.0)
        fx = gx.astype(jnp.int32)
        fy = gy.astype(jnp.int32)
        gi = jnp.clip(fx, 0, _W - 1)
        gj = jnp.clip(fy, 0, _H - 1)
        row = ((bi * _A + a) * _H + gj) * _W + gi
        tbx = gx - fx.astype(jnp.float32)
        tby = gy - fy.astype(jnp.float32)
        return (mask.astype(jnp.float32), row, tbx, tby, gw, gh, cls,
                jnp.full((_NT, 1), anc_w, jnp.float32),
                jnp.full((_NT, 1), anc_h, jnp.float32))

    def catpad(parts, padval, dtype):
        seg = jnp.full((_NTP - _NT, 1), padval, dtype)
        tail = jnp.full((_MP - _A * _NTP, 1), padval, dtype)
        out = []
        for p in parts:
            out += [p, seg]
        return jnp.concatenate(out + [tail], axis=0)

    c0, c1, c2 = cand_cols(0), cand_cols(1), cand_cols(2)
    mf = catpad([c0[0], c1[0], c2[0]], 0.0, jnp.float32)
    row = catpad([c0[1], c1[1], c2[1]], -1, jnp.int32)
    tbx = catpad([c0[2], c1[2], c2[2]], 0.0, jnp.float32)
    tby = catpad([c0[3], c1[3], c2[3]], 0.0, jnp.float32)
    tbw = catpad([c0[4], c1[4], c2[4]], 0.0, jnp.float32)
    tbh = catpad([c0[5], c1[5], c2[5]], 0.0, jnp.float32)
    cls = catpad([c0[6], c1[6], c2[6]], 0, jnp.int32)
    anw = catpad([c0[7], c1[7], c2[7]], 1.0, jnp.float32)
    anh = catpad([c0[8], c1[8], c2[8]], 1.0, jnp.float32)

    def cand_rows(a):
        anc_w = anch_ref[a, 0] / s
        anc_h = anch_ref[a, 1] / s
        bi = tt_ref[0:1, :].astype(jnp.int32)
        gx = tt_ref[2:3, :] * gain
        gy = tt_ref[3:4, :] * gain
        gw = tt_ref[4:5, :] * gain
        gh = tt_ref[5:6, :] * gain
        rw = gw / anc_w
        rh = gh / anc_h
        mask = jnp.logical_and(jnp.maximum(rw, 1.0 / rw) < 4.0,
                               jnp.maximum(rh, 1.0 / rh) < 4.0)
        gi = jnp.clip(gx.astype(jnp.int32), 0, _W - 1)
        gj = jnp.clip(gy.astype(jnp.int32), 0, _H - 1)
        rowr = ((bi * _A + a) * _H + gj) * _W + gi
        return mask.astype(jnp.float32), rowr

    r0, r1, r2 = cand_rows(0), cand_rows(1), cand_rows(2)
    padm = jnp.zeros((1, _NTP - _NT), jnp.float32)
    padr = jnp.full((1, _NTP - _NT), -2, jnp.int32)
    tailm = jnp.zeros((1, _MP - _A * _NTP), jnp.float32)
    tailr = jnp.full((1, _MP - _A * _NTP), -2, jnp.int32)
    mf_r = jnp.concatenate([r0[0], padm, r1[0], padm, r2[0], padm, tailm],
                           axis=1)
    row_r = jnp.concatenate([r0[1], padr, r1[1], padr, r2[1], padr, tailr],
                            axis=1)

    kk = lax.broadcasted_iota(jnp.int32, (_MP, _MP), 0)
    jj = lax.broadcasted_iota(jnp.int32, (_MP, _MP), 1)
    later_dup = (row == row_r).astype(jnp.float32) * mf_r * (jj > kk).astype(
        jnp.float32)
    ndup = jnp.sum(later_dup, axis=1, keepdims=True)
    winner = mf * (ndup < 0.5).astype(jnp.float32)

    pxy_x = 1.0 / (1.0 + jnp.exp(-ps_ref[:, 0:1]))
    pxy_y = 1.0 / (1.0 + jnp.exp(-ps_ref[:, 1:2]))
    pw = jnp.exp(ps_ref[:, 2:3]) * anw
    ph = jnp.exp(ps_ref[:, 3:4]) * anh
    p4 = ps_ref[:, 4:5]

    b1x1 = pxy_x - pw * 0.5
    b1x2 = pxy_x + pw * 0.5
    b1y1 = pxy_y - ph * 0.5
    b1y2 = pxy_y + ph * 0.5
    b2x1 = tbx - tbw * 0.5
    b2x2 = tbx + tbw * 0.5
    b2y1 = tby - tbh * 0.5
    b2y2 = tby + tbh * 0.5
    iw = jnp.maximum(jnp.minimum(b1x2, b2x2) - jnp.maximum(b1x1, b2x1), 0.0)
    ih = jnp.maximum(jnp.minimum(b1y2, b2y2) - jnp.maximum(b1y1, b2y1), 0.0)
    inter = iw * ih
    union = pw * ph + tbw * tbh - inter + 1e-9
    iou = inter / union

    msum = jnp.sum(mf)
    denom = jnp.maximum(msum, 1.0)
    has = (msum > 0.0).astype(jnp.float32)
    lbox = has * jnp.sum((1.0 - iou) * mf) / denom

    logits = ps_ref[:, 5:_C]
    cc = lax.broadcasted_iota(jnp.int32, (_MP, _NC), 1)
    sel = jnp.sum(logits * (cc == cls).astype(jnp.float32), axis=1,
                  keepdims=True)
    spsum = jnp.sum(_softplus(logits), axis=1, keepdims=True)
    lcls = has * jnp.sum((spsum - sel) * mf) / (denom * _NC)

    objsum = jnp.sum(_softplus(obj_ref[...]))
    corr = jnp.sum(winner * p4 * jnp.maximum(iou, 0.0))
    lobj = (objsum - corr) / jnp.float32(_ROWS)

    o_ref[0, 0] = 0.05 * lcls + lobj + 0.5 * lbox


@jax.jit
def kernel(pred, targets, anchors, stride):
    predflat = pred.reshape(_ROWS * _C)
    targt = jnp.zeros((6, _NTP), jnp.float32).at[:, :_NT].set(targets.T)
    obj, rows = _sc_gather(predflat, targt.reshape(-1))

    loss = pl.pallas_call(
        _loss_body,
        in_specs=[
            pl.BlockSpec(memory_space=pltpu.VMEM),
            pl.BlockSpec(memory_space=pltpu.VMEM),
            pl.BlockSpec(memory_space=pltpu.VMEM),
            pl.BlockSpec(memory_space=pltpu.VMEM),
            pl.BlockSpec(memory_space=pltpu.SMEM),
            pl.BlockSpec(memory_space=pltpu.SMEM),
        ],
        out_specs=pl.BlockSpec(memory_space=pltpu.SMEM),
        out_shape=jax.ShapeDtypeStruct((1, 1), jnp.float32),
    )(obj.reshape(_ROWS // 128, 128), rows.reshape(_MP, _CW),
      targets, targets.T, anchors, stride.reshape(1))

    return loss.reshape(())


# speedup vs baseline: 2.2073x; 1.1297x over previous
import functools

import jax
import jax.numpy as jnp
from jax import lax
from jax.experimental import pallas as pl
from jax.experimental.pallas import tpu as pltpu
from jax.experimental.pallas import tpu_sc as plsc

_B, _A, _H, _W, _NC = 16, 3, 80, 80, 80
_NT = 200
_NTP = 208
_M = _A * _NT
_MP = 640
_ROWS = _B * _A * _H * _W
_C = 5 + _NC

_NTILES = 32
_NGRP = _MP // 16
_BLK = 2048
_NBLK = _ROWS // _BLK


def _softplus(x):
    return jnp.maximum(x, 0.0) + jnp.log1p(jnp.exp(-jnp.abs(x)))


def _step01(x):
    return jnp.minimum(jnp.maximum(x, 0), 1)


def _sc_body(targ_hbm, ridx_hbm, tvm, rvbuf, sem_t):
    wid = lax.axis_index("s") * 2 + lax.axis_index("c")
    lanes = lax.iota(jnp.int32, 16)

    pltpu.async_copy(targ_hbm, tvm, sem_t).wait()

    def do_group(gg, slot):
        a3 = (_step01(gg - 12) + _step01(gg - 25) + _step01(gg - 38))
        a_c = jnp.minimum(a3, 2)
        i0 = (gg - 13 * a3) * 16
        bf = tvm[pl.ds(i0, 16)]
        xf = tvm[pl.ds(2 * _NTP + i0, 16)]
        yf = tvm[pl.ds(3 * _NTP + i0, 16)]
        gi = jnp.clip((xf * jnp.float32(_W)).astype(jnp.int32), 0, _W - 1)
        gj = jnp.clip((yf * jnp.float32(_H)).astype(jnp.int32), 0, _H - 1)
        bi = bf.astype(jnp.int32)
        rvbuf[pl.ds(slot * 16, 16)] = ((bi * _A + a_c) * _H + gj) * _W + gi

    do_group(wid, 0)

    @pl.when(wid < _NGRP - _NTILES)
    def _():
        do_group(wid + _NTILES, 1)

    pltpu.sync_copy(rvbuf.at[pl.ds(0, 16)],
                    ridx_hbm.at[pl.ds(wid * 16, 16)])

    @pl.when(wid < _NGRP - _NTILES)
    def _():
        pltpu.sync_copy(rvbuf.at[pl.ds(16, 16)],
                        ridx_hbm.at[pl.ds((wid + _NTILES) * 16, 16)])


def _sc_route(targt):
    mesh = plsc.VectorSubcoreMesh(core_axis_name="c", subcore_axis_name="s")
    return pl.kernel(
        _sc_body,
        out_type=jax.ShapeDtypeStruct((_MP,), jnp.int32),
        mesh=mesh,
        scratch_types=[
            pltpu.VMEM((6 * _NTP,), jnp.float32),
            pltpu.VMEM((32,), jnp.int32),
            pltpu.SemaphoreType.DMA,
        ],
    )(targt)


def _loss_body(ridx_ref, x_ref, pred_ref, t_ref, tt_ref, anch_ref, s_ref,
               o_ref, acc_ref, ps_ref, sem):
    i = pl.program_id(0)

    @pl.when(i == 0)
    def _():
        acc_ref[0] = 0.0

        def fire(k, carry):
            row = ridx_ref[k]
            pltpu.make_async_copy(pred_ref.at[pl.ds(row, 1), :],
                                  ps_ref.at[pl.ds(k, 1), :], sem).start()
            return carry

        lax.fori_loop(0, _MP, fire, 0)

    acc_ref[0] += jnp.sum(_softplus(x_ref[:, 4:5]))

    @pl.when(i == pl.num_programs(0) - 1)
    def _():
        def drain(k, carry):
            pltpu.make_async_copy(pred_ref.at[pl.ds(0, 1), :],
                                  ps_ref.at[pl.ds(k, 1), :], sem).wait()
            return carry

        lax.fori_loop(0, _MP, drain, 0)

        s = s_ref[0]
        gain = jnp.float32(_W)

        def cand_cols(a):
            anc_w = anch_ref[a, 0] / s
            anc_h = anch_ref[a, 1] / s
            bi = t_ref[:, 0:1].astype(jnp.int32)
            cls = t_ref[:, 1:2].astype(jnp.int32)
            gx = t_ref[:, 2:3] * gain
            gy = t_ref[:, 3:4] * gain
            gw = t_ref[:, 4:5] * gain
            gh = t_ref[:, 5:6] * gain
            rw = gw / anc_w
            rh = gh / anc_h
            mask = jnp.logical_and(jnp.maximum(rw, 1.0 / rw) < 4.0,
                                   jnp.maximum(rh, 1.0 / rh) < 4.0)
            fx = gx.astype(jnp.int32)
            fy = gy.astype(jnp.int32)
            gi = jnp.clip(fx, 0, _W - 1)
            gj = jnp.clip(fy, 0, _H - 1)
            row = ((bi * _A + a) * _H + gj) * _W + gi
            tbx = gx - fx.astype(jnp.float32)
            tby = gy - fy.astype(jnp.float32)
            return (mask.astype(jnp.float32), row, tbx, tby, gw, gh, cls,
                    jnp.full((_NT, 1), anc_w, jnp.float32),
                    jnp.full((_NT, 1), anc_h, jnp.float32))

        def catpad(parts, padval, dtype):
            seg = jnp.full((_NTP - _NT, 1), padval, dtype)
            tail = jnp.full((_MP - _A * _NTP, 1), padval, dtype)
            out = []
            for p in parts:
                out += [p, seg]
            return jnp.concatenate(out + [tail], axis=0)

        c0, c1, c2 = cand_cols(0), cand_cols(1), cand_cols(2)
        mf = catpad([c0[0], c1[0], c2[0]], 0.0, jnp.float32)
        row = catpad([c0[1], c1[1], c2[1]], -1, jnp.int32)
        tbx = catpad([c0[2], c1[2], c2[2]], 0.0, jnp.float32)
        tby = catpad([c0[3], c1[3], c2[3]], 0.0, jnp.float32)
        tbw = catpad([c0[4], c1[4], c2[4]], 0.0, jnp.float32)
        tbh = catpad([c0[5], c1[5], c2[5]], 0.0, jnp.float32)
        cls = catpad([c0[6], c1[6], c2[6]], 0, jnp.int32)
        anw = catpad([c0[7], c1[7], c2[7]], 1.0, jnp.float32)
        anh = catpad([c0[8], c1[8], c2[8]], 1.0, jnp.float32)

        def cand_rows(a):
            anc_w = anch_ref[a, 0] / s
            anc_h = anch_ref[a, 1] / s
            bi = tt_ref[0:1, :].astype(jnp.int32)
            gx = tt_ref[2:3, :] * gain
            gy = tt_ref[3:4, :] * gain
            gw = tt_ref[4:5, :] * gain
            gh = tt_ref[5:6, :] * gain
            rw = gw / anc_w
            rh = gh / anc_h
            mask = jnp.logical_and(jnp.maximum(rw, 1.0 / rw) < 4.0,
                                   jnp.maximum(rh, 1.0 / rh) < 4.0)
            gi = jnp.clip(gx.astype(jnp.int32), 0, _W - 1)
            gj = jnp.clip(gy.astype(jnp.int32), 0, _H - 1)
            rowr = ((bi * _A + a) * _H + gj) * _W + gi
            return mask.astype(jnp.float32), rowr

        r0, r1, r2 = cand_rows(0), cand_rows(1), cand_rows(2)
        padm = jnp.zeros((1, _NTP - _NT), jnp.float32)
        padr = jnp.full((1, _NTP - _NT), -2, jnp.int32)
        tailm = jnp.zeros((1, _MP - _A * _NTP), jnp.float32)
        tailr = jnp.full((1, _MP - _A * _NTP), -2, jnp.int32)
        mf_r = jnp.concatenate(
            [r0[0], padm, r1[0], padm, r2[0], padm, tailm], axis=1)
        row_r = jnp.concatenate(
            [r0[1], padr, r1[1], padr, r2[1], padr, tailr], axis=1)

        kk = lax.broadcasted_iota(jnp.int32, (_MP, _MP), 0)
        jj = lax.broadcasted_iota(jnp.int32, (_MP, _MP), 1)
        later_dup = ((row == row_r).astype(jnp.float32) * mf_r
                     * (jj > kk).astype(jnp.float32))
        ndup = jnp.sum(later_dup, axis=1, keepdims=True)
        winner = mf * (ndup < 0.5).astype(jnp.float32)

        pxy_x = 1.0 / (1.0 + jnp.exp(-ps_ref[:, 0:1]))
        pxy_y = 1.0 / (1.0 + jnp.exp(-ps_ref[:, 1:2]))
        pw = jnp.exp(ps_ref[:, 2:3]) * anw
        ph = jnp.exp(ps_ref[:, 3:4]) * anh
        p4 = ps_ref[:, 4:5]

        b1x1 = pxy_x - pw * 0.5
        b1x2 = pxy_x + pw * 0.5
        b1y1 = pxy_y - ph * 0.5
        b1y2 = pxy_y + ph * 0.5
        b2x1 = tbx - tbw * 0.5
        b2x2 = tbx + tbw * 0.5
        b2y1 = tby - tbh * 0.5
        b2y2 = tby + tbh * 0.5
        iw = jnp.maximum(
            jnp.minimum(b1x2, b2x2) - jnp.maximum(b1x1, b2x1), 0.0)
        ih = jnp.maximum(
            jnp.minimum(b1y2, b2y2) - jnp.maximum(b1y1, b2y1), 0.0)
        inter = iw * ih
        union = pw * ph + tbw * tbh - inter + 1e-9
        iou = inter / union

        msum = jnp.sum(mf)
        denom = jnp.maximum(msum, 1.0)
        has = (msum > 0.0).astype(jnp.float32)
        lbox = has * jnp.sum((1.0 - iou) * mf) / denom

        logits = ps_ref[:, 5:_C]
        cc = lax.broadcasted_iota(jnp.int32, (_MP, _NC), 1)
        sel = jnp.sum(logits * (cc == cls).astype(jnp.float32), axis=1,
                      keepdims=True)
        spsum = jnp.sum(_softplus(logits), axis=1, keepdims=True)
        lcls = has * jnp.sum((spsum - sel) * mf) / (denom * _NC)

        corr = jnp.sum(winner * p4 * jnp.maximum(iou, 0.0))
        lobj = (acc_ref[0] - corr) / jnp.float32(_ROWS)

        o_ref[0, 0] = 0.05 * lcls + lobj + 0.5 * lbox


@jax.jit
def kernel(pred, targets, anchors, stride):
    pred2d = pred.reshape(_ROWS, _C)
    targt = jnp.zeros((6, _NTP), jnp.float32).at[:, :_NT].set(targets.T)
    ridx = _sc_route(targt.reshape(-1))

    loss = pl.pallas_call(
        _loss_body,
        grid_spec=pltpu.PrefetchScalarGridSpec(
            num_scalar_prefetch=1,
            grid=(_NBLK,),
            in_specs=[
                pl.BlockSpec((_BLK, _C), lambda i, r: (i, 0)),
                pl.BlockSpec(memory_space=pl.ANY),
                pl.BlockSpec(memory_space=pltpu.VMEM),
                pl.BlockSpec(memory_space=pltpu.VMEM),
                pl.BlockSpec(memory_space=pltpu.SMEM),
                pl.BlockSpec(memory_space=pltpu.SMEM),
            ],
            out_specs=pl.BlockSpec(memory_space=pltpu.SMEM),
            scratch_shapes=[
                pltpu.SMEM((1,), jnp.float32),
                pltpu.VMEM((_MP, _C), jnp.float32),
                pltpu.SemaphoreType.DMA,
            ],
        ),
        out_shape=jax.ShapeDtypeStruct((1, 1), jnp.float32),
    )(ridx, pred2d, pred2d, targets, targets.T, anchors, stride.reshape(1))

    return loss.reshape(())


# speedup vs baseline: 2.8476x; 1.2901x over previous
import functools

import jax
import jax.numpy as jnp
from jax import lax
from jax.experimental import pallas as pl
from jax.experimental.pallas import tpu as pltpu
from jax.experimental.pallas import tpu_sc as plsc

_B, _A, _H, _W, _NC = 16, 3, 80, 80, 80
_NT = 200
_NTP = 208
_M = _A * _NT
_MP = 640
_ROWS = _B * _A * _H * _W
_C = 5 + _NC

_NTILES = 32
_NGRP = _MP // 16
_BLK = 4096
_NBLK = _ROWS // _BLK


def _softplus(x):
    return jnp.maximum(x, 0.0) + jnp.log1p(jnp.exp(-jnp.abs(x)))


def _step01(x):
    return jnp.minimum(jnp.maximum(x, 0), 1)


def _sc_body(targ_hbm, ridx_hbm, tvm, rvbuf, sem_t):
    wid = lax.axis_index("s") * 2 + lax.axis_index("c")
    lanes = lax.iota(jnp.int32, 16)

    pltpu.async_copy(targ_hbm, tvm, sem_t).wait()

    def do_group(gg, slot):
        a3 = (_step01(gg - 12) + _step01(gg - 25) + _step01(gg - 38))
        a_c = jnp.minimum(a3, 2)
        i0 = (gg - 13 * a3) * 16
        bf = tvm[pl.ds(i0, 16)]
        xf = tvm[pl.ds(2 * _NTP + i0, 16)]
        yf = tvm[pl.ds(3 * _NTP + i0, 16)]
        gi = jnp.clip((xf * jnp.float32(_W)).astype(jnp.int32), 0, _W - 1)
        gj = jnp.clip((yf * jnp.float32(_H)).astype(jnp.int32), 0, _H - 1)
        bi = bf.astype(jnp.int32)
        rvbuf[pl.ds(slot * 16, 16)] = ((bi * _A + a_c) * _H + gj) * _W + gi

    do_group(wid, 0)

    @pl.when(wid < _NGRP - _NTILES)
    def _():
        do_group(wid + _NTILES, 1)

    pltpu.sync_copy(rvbuf.at[pl.ds(0, 16)],
                    ridx_hbm.at[pl.ds(wid * 16, 16)])

    @pl.when(wid < _NGRP - _NTILES)
    def _():
        pltpu.sync_copy(rvbuf.at[pl.ds(16, 16)],
                        ridx_hbm.at[pl.ds((wid + _NTILES) * 16, 16)])


def _sc_route(targt):
    mesh = plsc.VectorSubcoreMesh(core_axis_name="c", subcore_axis_name="s")
    return pl.kernel(
        _sc_body,
        out_type=jax.ShapeDtypeStruct((_MP,), jnp.int32),
        mesh=mesh,
        scratch_types=[
            pltpu.VMEM((6 * _NTP,), jnp.float32),
            pltpu.VMEM((32,), jnp.int32),
            pltpu.SemaphoreType.DMA,
        ],
    )(targt)


def _loss_body(ridx_ref, x_ref, pred_ref, t_ref, tt_ref, anch_ref, s_ref,
               o_ref, acc_ref, ps_ref, sem):
    i = pl.program_id(0)

    @pl.when(i == 0)
    def _():
        acc_ref[0] = 0.0

        def fire(k, carry):
            row = ridx_ref[k]
            pltpu.make_async_copy(pred_ref.at[pl.ds(row, 1), :],
                                  ps_ref.at[pl.ds(k, 1), :], sem).start()
            return carry

        lax.fori_loop(0, _MP, fire, 0)

    acc_ref[0] += jnp.sum(_softplus(x_ref[:, 4:5]))

    @pl.when(i == pl.num_programs(0) - 1)
    def _():
        def drain(k, carry):
            pltpu.make_async_copy(pred_ref.at[pl.ds(0, 1), :],
                                  ps_ref.at[pl.ds(k, 1), :], sem).wait()
            return carry

        lax.fori_loop(0, _MP, drain, 0)

        s = s_ref[0]
        gain = jnp.float32(_W)

        def cand_cols(a):
            anc_w = anch_ref[a, 0] / s
            anc_h = anch_ref[a, 1] / s
            bi = t_ref[:, 0:1].astype(jnp.int32)
            cls = t_ref[:, 1:2].astype(jnp.int32)
            gx = t_ref[:, 2:3] * gain
            gy = t_ref[:, 3:4] * gain
            gw = t_ref[:, 4:5] * gain
            gh = t_ref[:, 5:6] * gain
            rw = gw / anc_w
            rh = gh / anc_h
            mask = jnp.logical_and(jnp.maximum(rw, 1.0 / rw) < 4.0,
                                   jnp.maximum(rh, 1.0 / rh) < 4.0)
            fx = gx.astype(jnp.int32)
            fy = gy.astype(jnp.int32)
            gi = jnp.clip(fx, 0, _W - 1)
            gj = jnp.clip(fy, 0, _H - 1)
            row = ((bi * _A + a) * _H + gj) * _W + gi
            tbx = gx - fx.astype(jnp.float32)
            tby = gy - fy.astype(jnp.float32)
            return (mask.astype(jnp.float32), row, tbx, tby, gw, gh, cls,
                    jnp.full((_NT, 1), anc_w, jnp.float32),
                    jnp.full((_NT, 1), anc_h, jnp.float32))

        def catpad(parts, padval, dtype):
            seg = jnp.full((_NTP - _NT, 1), padval, dtype)
            tail = jnp.full((_MP - _A * _NTP, 1), padval, dtype)
            out = []
            for p in parts:
                out += [p, seg]
            return jnp.concatenate(out + [tail], axis=0)

        c0, c1, c2 = cand_cols(0), cand_cols(1), cand_cols(2)
        mf = catpad([c0[0], c1[0], c2[0]], 0.0, jnp.float32)
        row = catpad([c0[1], c1[1], c2[1]], -1, jnp.int32)
        tbx = catpad([c0[2], c1[2], c2[2]], 0.0, jnp.float32)
        tby = catpad([c0[3], c1[3], c2[3]], 0.0, jnp.float32)
        tbw = catpad([c0[4], c1[4], c2[4]], 0.0, jnp.float32)
        tbh = catpad([c0[5], c1[5], c2[5]], 0.0, jnp.float32)
        cls = catpad([c0[6], c1[6], c2[6]], 0, jnp.int32)
        anw = catpad([c0[7], c1[7], c2[7]], 1.0, jnp.float32)
        anh = catpad([c0[8], c1[8], c2[8]], 1.0, jnp.float32)

        def cand_rows(a):
            anc_w = anch_ref[a, 0] / s
            anc_h = anch_ref[a, 1] / s
            bi = tt_ref[0:1, :].astype(jnp.int32)
            gx = tt_ref[2:3, :] * gain
            gy = tt_ref[3:4, :] * gain
            gw = tt_ref[4:5, :] * gain
            gh = tt_ref[5:6, :] * gain
            rw = gw / anc_w
            rh = gh / anc_h
            mask = jnp.logical_and(jnp.maximum(rw, 1.0 / rw) < 4.0,
                                   jnp.maximum(rh, 1.0 / rh) < 4.0)
            gi = jnp.clip(gx.astype(jnp.int32), 0, _W - 1)
            gj = jnp.clip(gy.astype(jnp.int32), 0, _H - 1)
            rowr = ((bi * _A + a) * _H + gj) * _W + gi
            return mask.astype(jnp.float32), rowr

        r0, r1, r2 = cand_rows(0), cand_rows(1), cand_rows(2)
        padm = jnp.zeros((1, _NTP - _NT), jnp.float32)
        padr = jnp.full((1, _NTP - _NT), -2, jnp.int32)
        tailm = jnp.zeros((1, _MP - _A * _NTP), jnp.float32)
        tailr = jnp.full((1, _MP - _A * _NTP), -2, jnp.int32)
        mf_r = jnp.concatenate(
            [r0[0], padm, r1[0], padm, r2[0], padm, tailm], axis=1)
        row_r = jnp.concatenate(
            [r0[1], padr, r1[1], padr, r2[1], padr, tailr], axis=1)

        kk = lax.broadcasted_iota(jnp.int32, (_MP, _MP), 0)
        jj = lax.broadcasted_iota(jnp.int32, (_MP, _MP), 1)
        later_dup = ((row == row_r).astype(jnp.float32) * mf_r
                     * (jj > kk).astype(jnp.float32))
        ndup = jnp.sum(later_dup, axis=1, keepdims=True)
        winner = mf * (ndup < 0.5).astype(jnp.float32)

        pxy_x = 1.0 / (1.0 + jnp.exp(-ps_ref[:, 0:1]))
        pxy_y = 1.0 / (1.0 + jnp.exp(-ps_ref[:, 1:2]))
        pw = jnp.exp(ps_ref[:, 2:3]) * anw
        ph = jnp.exp(ps_ref[:, 3:4]) * anh
        p4 = ps_ref[:, 4:5]

        b1x1 = pxy_x - pw * 0.5
        b1x2 = pxy_x + pw * 0.5
        b1y1 = pxy_y - ph * 0.5
        b1y2 = pxy_y + ph * 0.5
        b2x1 = tbx - tbw * 0.5
        b2x2 = tbx + tbw * 0.5
        b2y1 = tby - tbh * 0.5
        b2y2 = tby + tbh * 0.5
        iw = jnp.maximum(
            jnp.minimum(b1x2, b2x2) - jnp.maximum(b1x1, b2x1), 0.0)
        ih = jnp.maximum(
            jnp.minimum(b1y2, b2y2) - jnp.maximum(b1y1, b2y1), 0.0)
        inter = iw * ih
        union = pw * ph + tbw * tbh - inter + 1e-9
        iou = inter / union

        msum = jnp.sum(mf)
        denom = jnp.maximum(msum, 1.0)
        has = (msum > 0.0).astype(jnp.float32)
        lbox = has * jnp.sum((1.0 - iou) * mf) / denom

        logits = ps_ref[:, 5:_C]
        cc = lax.broadcasted_iota(jnp.int32, (_MP, _NC), 1)
        sel = jnp.sum(logits * (cc == cls).astype(jnp.float32), axis=1,
                      keepdims=True)
        spsum = jnp.sum(_softplus(logits), axis=1, keepdims=True)
        lcls = has * jnp.sum((spsum - sel) * mf) / (denom * _NC)

        corr = jnp.sum(winner * p4 * jnp.maximum(iou, 0.0))
        lobj = (acc_ref[0] - corr) / jnp.float32(_ROWS)

        o_ref[0, 0] = 0.05 * lcls + lobj + 0.5 * lbox


@jax.jit
def kernel(pred, targets, anchors, stride):
    pred2d = pred.reshape(_ROWS, _C)
    targt = jnp.zeros((6, _NTP), jnp.float32).at[:, :_NT].set(targets.T)
    ridx = _sc_route(targt.reshape(-1))

    loss = pl.pallas_call(
        _loss_body,
        grid_spec=pltpu.PrefetchScalarGridSpec(
            num_scalar_prefetch=1,
            grid=(_NBLK,),
            in_specs=[
                pl.BlockSpec((_BLK, _C), lambda i, r: (i, 0)),
                pl.BlockSpec(memory_space=pl.ANY),
                pl.BlockSpec(memory_space=pltpu.VMEM),
                pl.BlockSpec(memory_space=pltpu.VMEM),
                pl.BlockSpec(memory_space=pltpu.SMEM),
                pl.BlockSpec(memory_space=pltpu.SMEM),
            ],
            out_specs=pl.BlockSpec(memory_space=pltpu.SMEM),
            scratch_shapes=[
                pltpu.SMEM((1,), jnp.float32),
                pltpu.VMEM((_MP, _C), jnp.float32),
                pltpu.SemaphoreType.DMA,
            ],
        ),
        out_shape=jax.ShapeDtypeStruct((1, 1), jnp.float32),
    )(ridx, pred2d, pred2d, targets, targets.T, anchors, stride.reshape(1))

    return loss.reshape(())


# speedup vs baseline: 3.2486x; 1.1408x over previous
import functools

import jax
import jax.numpy as jnp
from jax import lax
from jax.experimental import pallas as pl
from jax.experimental.pallas import tpu as pltpu
from jax.experimental.pallas import tpu_sc as plsc

_B, _A, _H, _W, _NC = 16, 3, 80, 80, 80
_NT = 200
_NTP = 208
_M = _A * _NT
_MP = 640
_ROWS = _B * _A * _H * _W
_C = 5 + _NC

_NTILES = 32
_NGRP = _MP // 16
_BLK = 12288
_NBLK = _ROWS // _BLK


def _softplus(x):
    return jnp.maximum(x, 0.0) + jnp.log1p(jnp.exp(-jnp.abs(x)))


def _step01(x):
    return jnp.minimum(jnp.maximum(x, 0), 1)


def _sc_body(targ_hbm, ridx_hbm, tvm, rvbuf, sem_t):
    wid = lax.axis_index("s") * 2 + lax.axis_index("c")
    lanes = lax.iota(jnp.int32, 16)

    pltpu.async_copy(targ_hbm, tvm, sem_t).wait()

    def do_group(gg, slot):
        a3 = (_step01(gg - 12) + _step01(gg - 25) + _step01(gg - 38))
        a_c = jnp.minimum(a3, 2)
        i0 = (gg - 13 * a3) * 16
        bf = tvm[pl.ds(i0, 16)]
        xf = tvm[pl.ds(2 * _NTP + i0, 16)]
        yf = tvm[pl.ds(3 * _NTP + i0, 16)]
        gi = jnp.clip((xf * jnp.float32(_W)).astype(jnp.int32), 0, _W - 1)
        gj = jnp.clip((yf * jnp.float32(_H)).astype(jnp.int32), 0, _H - 1)
        bi = bf.astype(jnp.int32)
        rvbuf[pl.ds(slot * 16, 16)] = ((bi * _A + a_c) * _H + gj) * _W + gi

    do_group(wid, 0)

    @pl.when(wid < _NGRP - _NTILES)
    def _():
        do_group(wid + _NTILES, 1)

    pltpu.sync_copy(rvbuf.at[pl.ds(0, 16)],
                    ridx_hbm.at[pl.ds(wid * 16, 16)])

    @pl.when(wid < _NGRP - _NTILES)
    def _():
        pltpu.sync_copy(rvbuf.at[pl.ds(16, 16)],
                        ridx_hbm.at[pl.ds((wid + _NTILES) * 16, 16)])


def _sc_route(targt):
    mesh = plsc.VectorSubcoreMesh(core_axis_name="c", subcore_axis_name="s")
    return pl.kernel(
        _sc_body,
        out_type=jax.ShapeDtypeStruct((_MP,), jnp.int32),
        mesh=mesh,
        scratch_types=[
            pltpu.VMEM((6 * _NTP,), jnp.float32),
            pltpu.VMEM((32,), jnp.int32),
            pltpu.SemaphoreType.DMA,
        ],
    )(targt)


def _loss_body(ridx_ref, x_ref, pred_ref, t_ref, tt_ref, anch_ref, s_ref,
               o_ref, acc_ref, ps_ref, sem):
    i = pl.program_id(0)

    @pl.when(i == 0)
    def _():
        acc_ref[0] = 0.0

        def fire(k, carry):
            row = ridx_ref[k]
            pltpu.make_async_copy(pred_ref.at[pl.ds(row, 1), :],
                                  ps_ref.at[pl.ds(k, 1), :], sem).start()
            return carry

        lax.fori_loop(0, _MP, fire, 0)

    acc_ref[0] += jnp.sum(_softplus(x_ref[:, 4:5]))

    @pl.when(i == pl.num_programs(0) - 1)
    def _():
        def drain(k, carry):
            pltpu.make_async_copy(pred_ref.at[pl.ds(0, 1), :],
                                  ps_ref.at[pl.ds(k, 1), :], sem).wait()
            return carry

        lax.fori_loop(0, _MP, drain, 0)

        s = s_ref[0]
        gain = jnp.float32(_W)

        def cand_cols(a):
            anc_w = anch_ref[a, 0] / s
            anc_h = anch_ref[a, 1] / s
            bi = t_ref[:, 0:1].astype(jnp.int32)
            cls = t_ref[:, 1:2].astype(jnp.int32)
            gx = t_ref[:, 2:3] * gain
            gy = t_ref[:, 3:4] * gain
            gw = t_ref[:, 4:5] * gain
            gh = t_ref[:, 5:6] * gain
            rw = gw / anc_w
            rh = gh / anc_h
            mask = jnp.logical_and(jnp.maximum(rw, 1.0 / rw) < 4.0,
                                   jnp.maximum(rh, 1.0 / rh) < 4.0)
            fx = gx.astype(jnp.int32)
            fy = gy.astype(jnp.int32)
            gi = jnp.clip(fx, 0, _W - 1)
            gj = jnp.clip(fy, 0, _H - 1)
            row = ((bi * _A + a) * _H + gj) * _W + gi
            tbx = gx - fx.astype(jnp.float32)
            tby = gy - fy.astype(jnp.float32)
            return (mask.astype(jnp.float32), row, tbx, tby, gw, gh, cls,
                    jnp.full((_NT, 1), anc_w, jnp.float32),
                    jnp.full((_NT, 1), anc_h, jnp.float32))

        def catpad(parts, padval, dtype):
            seg = jnp.full((_NTP - _NT, 1), padval, dtype)
            tail = jnp.full((_MP - _A * _NTP, 1), padval, dtype)
            out = []
            for p in parts:
                out += [p, seg]
            return jnp.concatenate(out + [tail], axis=0)

        c0, c1, c2 = cand_cols(0), cand_cols(1), cand_cols(2)
        mf = catpad([c0[0], c1[0], c2[0]], 0.0, jnp.float32)
        row = catpad([c0[1], c1[1], c2[1]], -1, jnp.int32)
        tbx = catpad([c0[2], c1[2], c2[2]], 0.0, jnp.float32)
        tby = catpad([c0[3], c1[3], c2[3]], 0.0, jnp.float32)
        tbw = catpad([c0[4], c1[4], c2[4]], 0.0, jnp.float32)
        tbh = catpad([c0[5], c1[5], c2[5]], 0.0, jnp.float32)
        cls = catpad([c0[6], c1[6], c2[6]], 0, jnp.int32)
        anw = catpad([c0[7], c1[7], c2[7]], 1.0, jnp.float32)
        anh = catpad([c0[8], c1[8], c2[8]], 1.0, jnp.float32)

        def cand_rows(a):
            anc_w = anch_ref[a, 0] / s
            anc_h = anch_ref[a, 1] / s
            bi = tt_ref[0:1, :].astype(jnp.int32)
            gx = tt_ref[2:3, :] * gain
            gy = tt_ref[3:4, :] * gain
            gw = tt_ref[4:5, :] * gain
            gh = tt_ref[5:6, :] * gain
            rw = gw / anc_w
            rh = gh / anc_h
            mask = jnp.logical_and(jnp.maximum(rw, 1.0 / rw) < 4.0,
                                   jnp.maximum(rh, 1.0 / rh) < 4.0)
            gi = jnp.clip(gx.astype(jnp.int32), 0, _W - 1)
            gj = jnp.clip(gy.astype(jnp.int32), 0, _H - 1)
            rowr = ((bi * _A + a) * _H + gj) * _W + gi
            return mask.astype(jnp.float32), rowr

        r0, r1, r2 = cand_rows(0), cand_rows(1), cand_rows(2)
        padm = jnp.zeros((1, _NTP - _NT), jnp.float32)
        padr = jnp.full((1, _NTP - _NT), -2, jnp.int32)
        tailm = jnp.zeros((1, _MP - _A * _NTP), jnp.float32)
        tailr = jnp.full((1, _MP - _A * _NTP), -2, jnp.int32)
        mf_r = jnp.concatenate(
            [r0[0], padm, r1[0], padm, r2[0], padm, tailm], axis=1)
        row_r = jnp.concatenate(
            [r0[1], padr, r1[1], padr, r2[1], padr, tailr], axis=1)

        kk = lax.broadcasted_iota(jnp.int32, (_MP, _MP), 0)
        jj = lax.broadcasted_iota(jnp.int32, (_MP, _MP), 1)
        later_dup = ((row == row_r).astype(jnp.float32) * mf_r
                     * (jj > kk).astype(jnp.float32))
        ndup = jnp.sum(later_dup, axis=1, keepdims=True)
        winner = mf * (ndup < 0.5).astype(jnp.float32)

        pxy_x = 1.0 / (1.0 + jnp.exp(-ps_ref[:, 0:1]))
        pxy_y = 1.0 / (1.0 + jnp.exp(-ps_ref[:, 1:2]))
        pw = jnp.exp(ps_ref[:, 2:3]) * anw
        ph = jnp.exp(ps_ref[:, 3:4]) * anh
        p4 = ps_ref[:, 4:5]

        b1x1 = pxy_x - pw * 0.5
        b1x2 = pxy_x + pw * 0.5
        b1y1 = pxy_y - ph * 0.5
        b1y2 = pxy_y + ph * 0.5
        b2x1 = tbx - tbw * 0.5
        b2x2 = tbx + tbw * 0.5
        b2y1 = tby - tbh * 0.5
        b2y2 = tby + tbh * 0.5
        iw = jnp.maximum(
            jnp.minimum(b1x2, b2x2) - jnp.maximum(b1x1, b2x1), 0.0)
        ih = jnp.maximum(
            jnp.minimum(b1y2, b2y2) - jnp.maximum(b1y1, b2y1), 0.0)
        inter = iw * ih
        union = pw * ph + tbw * tbh - inter + 1e-9
        iou = inter / union

        msum = jnp.sum(mf)
        denom = jnp.maximum(msum, 1.0)
        has = (msum > 0.0).astype(jnp.float32)
        lbox = has * jnp.sum((1.0 - iou) * mf) / denom

        logits = ps_ref[:, 5:_C]
        cc = lax.broadcasted_iota(jnp.int32, (_MP, _NC), 1)
        sel = jnp.sum(logits * (cc == cls).astype(jnp.float32), axis=1,
                      keepdims=True)
        spsum = jnp.sum(_softplus(logits), axis=1, keepdims=True)
        lcls = has * jnp.sum((spsum - sel) * mf) / (denom * _NC)

        corr = jnp.sum(winner * p4 * jnp.maximum(iou, 0.0))
        lobj = (acc_ref[0] - corr) / jnp.float32(_ROWS)

        o_ref[0, 0] = 0.05 * lcls + lobj + 0.5 * lbox


@jax.jit
def kernel(pred, targets, anchors, stride):
    pred2d = pred.reshape(_ROWS, _C)
    targt = jnp.zeros((6, _NTP), jnp.float32).at[:, :_NT].set(targets.T)
    ridx = _sc_route(targt.reshape(-1))

    loss = pl.pallas_call(
        _loss_body,
        grid_spec=pltpu.PrefetchScalarGridSpec(
            num_scalar_prefetch=1,
            grid=(_NBLK,),
            in_specs=[
                pl.BlockSpec((_BLK, _C), lambda i, r: (i, 0)),
                pl.BlockSpec(memory_space=pl.ANY),
                pl.BlockSpec(memory_space=pltpu.VMEM),
                pl.BlockSpec(memory_space=pltpu.VMEM),
                pl.BlockSpec(memory_space=pltpu.SMEM),
                pl.BlockSpec(memory_space=pltpu.SMEM),
            ],
            out_specs=pl.BlockSpec(memory_space=pltpu.SMEM),
            scratch_shapes=[
                pltpu.SMEM((1,), jnp.float32),
                pltpu.VMEM((_MP, _C), jnp.float32),
                pltpu.SemaphoreType.DMA,
            ],
        ),
        out_shape=jax.ShapeDtypeStruct((1, 1), jnp.float32),
    )(ridx, pred2d, pred2d, targets, targets.T, anchors, stride.reshape(1))

    return loss.reshape(())


# speedup vs baseline: 3.2739x; 1.0078x over previous
import functools

import jax
import jax.numpy as jnp
from jax import lax
from jax.experimental import pallas as pl
from jax.experimental.pallas import tpu as pltpu
from jax.experimental.pallas import tpu_sc as plsc

_B, _A, _H, _W, _NC = 16, 3, 80, 80, 80
_NT = 200
_NTP = 208
_M = _A * _NT
_MP = 640
_ROWS = _B * _A * _H * _W
_C = 5 + _NC

_NTILES = 32
_NGRP = _MP // 16
_BLK = 30720
_NBLK = _ROWS // _BLK


def _softplus(x):
    return jnp.maximum(x, 0.0) + jnp.log1p(jnp.exp(-jnp.abs(x)))


def _step01(x):
    return jnp.minimum(jnp.maximum(x, 0), 1)


def _sc_body(targ_hbm, ridx_hbm, tvm, rvbuf, sem_t):
    wid = lax.axis_index("s") * 2 + lax.axis_index("c")
    lanes = lax.iota(jnp.int32, 16)

    pltpu.async_copy(targ_hbm, tvm, sem_t).wait()

    def do_group(gg, slot):
        a3 = (_step01(gg - 12) + _step01(gg - 25) + _step01(gg - 38))
        a_c = jnp.minimum(a3, 2)
        i0 = (gg - 13 * a3) * 16
        bf = tvm[pl.ds(i0, 16)]
        xf = tvm[pl.ds(2 * _NTP + i0, 16)]
        yf = tvm[pl.ds(3 * _NTP + i0, 16)]
        gi = jnp.clip((xf * jnp.float32(_W)).astype(jnp.int32), 0, _W - 1)
        gj = jnp.clip((yf * jnp.float32(_H)).astype(jnp.int32), 0, _H - 1)
        bi = bf.astype(jnp.int32)
        rvbuf[pl.ds(slot * 16, 16)] = ((bi * _A + a_c) * _H + gj) * _W + gi

    do_group(wid, 0)

    @pl.when(wid < _NGRP - _NTILES)
    def _():
        do_group(wid + _NTILES, 1)

    pltpu.sync_copy(rvbuf.at[pl.ds(0, 16)],
                    ridx_hbm.at[pl.ds(wid * 16, 16)])

    @pl.when(wid < _NGRP - _NTILES)
    def _():
        pltpu.sync_copy(rvbuf.at[pl.ds(16, 16)],
                        ridx_hbm.at[pl.ds((wid + _NTILES) * 16, 16)])


def _sc_route(targt):
    mesh = plsc.VectorSubcoreMesh(core_axis_name="c", subcore_axis_name="s")
    return pl.kernel(
        _sc_body,
        out_type=jax.ShapeDtypeStruct((_MP,), jnp.int32),
        mesh=mesh,
        scratch_types=[
            pltpu.VMEM((6 * _NTP,), jnp.float32),
            pltpu.VMEM((32,), jnp.int32),
            pltpu.SemaphoreType.DMA,
        ],
    )(targt)


def _loss_body(ridx_ref, x_ref, pred_ref, t_ref, tt_ref, anch_ref, s_ref,
               o_ref, acc_ref, ps_ref, sem):
    i = pl.program_id(0)

    @pl.when(i == 0)
    def _():
        acc_ref[0] = 0.0

        def fire(k, carry):
            row = ridx_ref[k]
            pltpu.make_async_copy(pred_ref.at[pl.ds(row, 1), :],
                                  ps_ref.at[pl.ds(k, 1), :], sem).start()
            return carry

        lax.fori_loop(0, _MP, fire, 0)

    acc_ref[0] += jnp.sum(_softplus(x_ref[:, 4:5]))

    @pl.when(i == pl.num_programs(0) - 1)
    def _():
        def drain(k, carry):
            pltpu.make_async_copy(pred_ref.at[pl.ds(0, 1), :],
                                  ps_ref.at[pl.ds(k, 1), :], sem).wait()
            return carry

        lax.fori_loop(0, _MP, drain, 0)

        s = s_ref[0]
        gain = jnp.float32(_W)

        def cand_cols(a):
            anc_w = anch_ref[a, 0] / s
            anc_h = anch_ref[a, 1] / s
            bi = t_ref[:, 0:1].astype(jnp.int32)
            cls = t_ref[:, 1:2].astype(jnp.int32)
            gx = t_ref[:, 2:3] * gain
            gy = t_ref[:, 3:4] * gain
            gw = t_ref[:, 4:5] * gain
            gh = t_ref[:, 5:6] * gain
            rw = gw / anc_w
            rh = gh / anc_h
            mask = jnp.logical_and(jnp.maximum(rw, 1.0 / rw) < 4.0,
                                   jnp.maximum(rh, 1.0 / rh) < 4.0)
            fx = gx.astype(jnp.int32)
            fy = gy.astype(jnp.int32)
            gi = jnp.clip(fx, 0, _W - 1)
            gj = jnp.clip(fy, 0, _H - 1)
            row = ((bi * _A + a) * _H + gj) * _W + gi
            tbx = gx - fx.astype(jnp.float32)
            tby = gy - fy.astype(jnp.float32)
            return (mask.astype(jnp.float32), row, tbx, tby, gw, gh, cls,
                    jnp.full((_NT, 1), anc_w, jnp.float32),
                    jnp.full((_NT, 1), anc_h, jnp.float32))

        def catpad(parts, padval, dtype):
            seg = jnp.full((_NTP - _NT, 1), padval, dtype)
            tail = jnp.full((_MP - _A * _NTP, 1), padval, dtype)
            out = []
            for p in parts:
                out += [p, seg]
            return jnp.concatenate(out + [tail], axis=0)

        c0, c1, c2 = cand_cols(0), cand_cols(1), cand_cols(2)
        mf = catpad([c0[0], c1[0], c2[0]], 0.0, jnp.float32)
        row = catpad([c0[1], c1[1], c2[1]], -1, jnp.int32)
        tbx = catpad([c0[2], c1[2], c2[2]], 0.0, jnp.float32)
        tby = catpad([c0[3], c1[3], c2[3]], 0.0, jnp.float32)
        tbw = catpad([c0[4], c1[4], c2[4]], 0.0, jnp.float32)
        tbh = catpad([c0[5], c1[5], c2[5]], 0.0, jnp.float32)
        cls = catpad([c0[6], c1[6], c2[6]], 0, jnp.int32)
        anw = catpad([c0[7], c1[7], c2[7]], 1.0, jnp.float32)
        anh = catpad([c0[8], c1[8], c2[8]], 1.0, jnp.float32)

        def cand_rows(a):
            anc_w = anch_ref[a, 0] / s
            anc_h = anch_ref[a, 1] / s
            bi = tt_ref[0:1, :].astype(jnp.int32)
            gx = tt_ref[2:3, :] * gain
            gy = tt_ref[3:4, :] * gain
            gw = tt_ref[4:5, :] * gain
            gh = tt_ref[5:6, :] * gain
            rw = gw / anc_w
            rh = gh / anc_h
            mask = jnp.logical_and(jnp.maximum(rw, 1.0 / rw) < 4.0,
                                   jnp.maximum(rh, 1.0 / rh) < 4.0)
            gi = jnp.clip(gx.astype(jnp.int32), 0, _W - 1)
            gj = jnp.clip(gy.astype(jnp.int32), 0, _H - 1)
            rowr = ((bi * _A + a) * _H + gj) * _W + gi
            return mask.astype(jnp.float32), rowr

        r0, r1, r2 = cand_rows(0), cand_rows(1), cand_rows(2)
        padm = jnp.zeros((1, _NTP - _NT), jnp.float32)
        padr = jnp.full((1, _NTP - _NT), -2, jnp.int32)
        tailm = jnp.zeros((1, _MP - _A * _NTP), jnp.float32)
        tailr = jnp.full((1, _MP - _A * _NTP), -2, jnp.int32)
        mf_r = jnp.concatenate(
            [r0[0], padm, r1[0], padm, r2[0], padm, tailm], axis=1)
        row_r = jnp.concatenate(
            [r0[1], padr, r1[1], padr, r2[1], padr, tailr], axis=1)

        kk = lax.broadcasted_iota(jnp.int32, (_MP, _MP), 0)
        jj = lax.broadcasted_iota(jnp.int32, (_MP, _MP), 1)
        later_dup = ((row == row_r).astype(jnp.float32) * mf_r
                     * (jj > kk).astype(jnp.float32))
        ndup = jnp.sum(later_dup, axis=1, keepdims=True)
        winner = mf * (ndup < 0.5).astype(jnp.float32)

        pxy_x = 1.0 / (1.0 + jnp.exp(-ps_ref[:, 0:1]))
        pxy_y = 1.0 / (1.0 + jnp.exp(-ps_ref[:, 1:2]))
        pw = jnp.exp(ps_ref[:, 2:3]) * anw
        ph = jnp.exp(ps_ref[:, 3:4]) * anh
        p4 = ps_ref[:, 4:5]

        b1x1 = pxy_x - pw * 0.5
        b1x2 = pxy_x + pw * 0.5
        b1y1 = pxy_y - ph * 0.5
        b1y2 = pxy_y + ph * 0.5
        b2x1 = tbx - tbw * 0.5
        b2x2 = tbx + tbw * 0.5
        b2y1 = tby - tbh * 0.5
        b2y2 = tby + tbh * 0.5
        iw = jnp.maximum(
            jnp.minimum(b1x2, b2x2) - jnp.maximum(b1x1, b2x1), 0.0)
        ih = jnp.maximum(
            jnp.minimum(b1y2, b2y2) - jnp.maximum(b1y1, b2y1), 0.0)
        inter = iw * ih
        union = pw * ph + tbw * tbh - inter + 1e-9
        iou = inter / union

        msum = jnp.sum(mf)
        denom = jnp.maximum(msum, 1.0)
        has = (msum > 0.0).astype(jnp.float32)
        lbox = has * jnp.sum((1.0 - iou) * mf) / denom

        logits = ps_ref[:, 5:_C]
        cc = lax.broadcasted_iota(jnp.int32, (_MP, _NC), 1)
        sel = jnp.sum(logits * (cc == cls).astype(jnp.float32), axis=1,
                      keepdims=True)
        spsum = jnp.sum(_softplus(logits), axis=1, keepdims=True)
        lcls = has * jnp.sum((spsum - sel) * mf) / (denom * _NC)

        corr = jnp.sum(winner * p4 * jnp.maximum(iou, 0.0))
        lobj = (acc_ref[0] - corr) / jnp.float32(_ROWS)

        o_ref[0, 0] = 0.05 * lcls + lobj + 0.5 * lbox


@jax.jit
def kernel(pred, targets, anchors, stride):
    pred2d = pred.reshape(_ROWS, _C)
    targt = jnp.zeros((6, _NTP), jnp.float32).at[:, :_NT].set(targets.T)
    ridx = _sc_route(targt.reshape(-1))

    loss = pl.pallas_call(
        _loss_body,
        grid_spec=pltpu.PrefetchScalarGridSpec(
            num_scalar_prefetch=1,
            grid=(_NBLK,),
            in_specs=[
                pl.BlockSpec((_BLK, _C), lambda i, r: (i, 0)),
                pl.BlockSpec(memory_space=pl.ANY),
                pl.BlockSpec(memory_space=pltpu.VMEM),
                pl.BlockSpec(memory_space=pltpu.VMEM),
                pl.BlockSpec(memory_space=pltpu.SMEM),
                pl.BlockSpec(memory_space=pltpu.SMEM),
            ],
            out_specs=pl.BlockSpec(memory_space=pltpu.SMEM),
            scratch_shapes=[
                pltpu.SMEM((1,), jnp.float32),
                pltpu.VMEM((_MP, _C), jnp.float32),
                pltpu.SemaphoreType.DMA,
            ],
        ),
        out_shape=jax.ShapeDtypeStruct((1, 1), jnp.float32),
    )(ridx, pred2d, pred2d, targets, targets.T, anchors, stride.reshape(1))

    return loss.reshape(())
